# Initial kernel scaffold; baseline (speedup 1.0000x reference)
import jax
import jax.numpy as jnp
from jax.experimental import pallas as pl


def kernel(x0, edge_attr0, edge_index0, x1, edge_attr1, edge_index1, batch0, batch1, W1A, b1A, W1B, b1B, W1C, b1C, W1D, b1D, W2, b2, W3, b3, Wm1, bm1, Wm2, bm2):
    raise NotImplementedError("write your pallas kernel here")


# speedup vs baseline: 1.2781x; 1.2781x over previous
import functools

import jax
import jax.numpy as jnp
from jax.experimental import pallas as pl

N = 10000
E = 320000
D = 128
NUM_GRAPHS = 64


def _agg(h, src, dst, coef, inv_deg):
    msg = h[src] * coef[:, None]
    y = jax.ops.segment_sum(msg, dst, num_segments=N)
    return y + h * inv_deg[:, None]


def _head_kernel(p_ref, wm1_ref, bm1_ref, wm2_ref, bm2_ref, out_ref):
    p = p_ref[...]
    h = jnp.maximum(p @ wm1_ref[...] + bm1_ref[...], 0.0)
    out_ref[...] = h @ wm2_ref[...] + bm2_ref[...]


def _branch(x, ea, ei, batch, W1A, b1A, W1B, b1B, W1C, b1C, W1D, b1D,
            W2, b2, W3, b3):
    src, dst = ei[0], ei[1]
    ws = [ea[:, 0], ea[:, 1], ea[:, 2], ea[:, 3]]
    degs = [jax.ops.segment_sum(w, dst, num_segments=N) + 1.0 for w in ws]
    deg1 = jax.ops.segment_sum(jnp.ones((E,), x.dtype), dst, num_segments=N) + 1.0
    diss = [deg ** -0.5 for deg in degs]
    dis1 = deg1 ** -0.5

    zs = []
    for w, deg, dis, W, b in zip(ws, degs, diss,
                                 (W1A, W1B, W1C, W1D), (b1A, b1B, b1C, b1D)):
        coef = dis[src] * w * dis[dst]
        y = _agg(x, src, dst, coef, 1.0 / deg)
        zs.append(jnp.maximum(y @ W + b, 0.0))
    z = jnp.concatenate(zs, axis=1)

    coef1 = dis1[src] * dis1[dst]
    h2 = z @ W2
    x2 = jnp.maximum(_agg(h2, src, dst, coef1, 1.0 / deg1) + b2, 0.0)
    h3 = x2 @ W3
    x3 = _agg(h3, src, dst, coef1, 1.0 / deg1) + b3

    cnt = jax.ops.segment_sum(jnp.ones((N,), x3.dtype), batch,
                              num_segments=NUM_GRAPHS)
    mean = jax.ops.segment_sum(x3, batch, num_segments=NUM_GRAPHS) \
        / jnp.maximum(cnt, 1.0)[:, None]
    mx = jax.ops.segment_max(x3, batch, num_segments=NUM_GRAPHS)
    return jnp.concatenate([mean, mx], axis=1)


def kernel(x0, edge_attr0, edge_index0, x1, edge_attr1, edge_index1,
           batch0, batch1,
           W1A, b1A, W1B, b1B, W1C, b1C, W1D, b1D, W2, b2, W3, b3,
           Wm1, bm1, Wm2, bm2):
    p0 = _branch(x0, edge_attr0, edge_index0, batch0,
                 W1A, b1A, W1B, b1B, W1C, b1C, W1D, b1D, W2, b2, W3, b3)
    p1 = _branch(x1, edge_attr1, edge_index1, batch1,
                 W1A, b1A, W1B, b1B, W1C, b1C, W1D, b1D, W2, b2, W3, b3)
    p = jnp.concatenate([p0, p1], axis=1)

    wm1p = jnp.pad(Wm1, ((0, 0), (0, 120)))
    bm1p = jnp.pad(bm1, (0, 120))
    wm2p = jnp.pad(jnp.pad(Wm2, ((0, 120), (0, 0))), ((0, 0), (0, 126)))
    bm2p = jnp.pad(bm2, (0, 126))
    out = pl.pallas_call(
        _head_kernel,
        out_shape=jax.ShapeDtypeStruct((NUM_GRAPHS, 128), jnp.float32),
    )(p, wm1p, bm1p, wm2p, bm2p)
    return out[:, :2]


# speedup vs baseline: 3.4707x; 2.7156x over previous
import functools

import jax
import jax.numpy as jnp
from jax import lax
from jax.experimental import pallas as pl
from jax.experimental.pallas import tpu as pltpu
from jax.experimental.pallas import tpu_sc as plsc

N = 10000
E = 320000
EALL = E + N
D = 128
G = 64
NC, NS, NT = 2, 16, 32
TB = 128
NB = 88
ET = NB * TB
EPAD = NT * ET
NP = 10240
RPT = NP // NS
NKW = 5


def _vsmesh():
    return plsc.VectorSubcoreMesh(
        core_axis_name="c", subcore_axis_name="s",
        num_cores=NC, num_subcores=NS)


def _splat(val):
    return jnp.zeros((16,), jnp.int32) + val


def _deg_body(dstT, wT5, out, dst_v, w_v, deg_t):
    c = lax.axis_index("c")
    s = lax.axis_index("s")
    wid = c * NS + s

    def zbody(i, _):
        deg_t[pl.ds(i * 16, 16)] = jnp.zeros((16,), jnp.float32)
        return 0
    lax.fori_loop(0, NKW * NP // 16, zbody, 0)

    pltpu.sync_copy(dstT.at[pl.ds(wid * NB, NB)], dst_v)
    for k in range(NKW):
        pltpu.sync_copy(wT5.at[pl.ds((wid * NKW + k) * NB, NB)], w_v)

        def jbody(j, _):
            for g in range(8):
                dv = dst_v[j, pl.ds(g * 16, 16)]
                wv = w_v[j, pl.ds(g * 16, 16)]
                plsc.addupdate_scatter(deg_t, [dv + k * NP], wv)
            return 0
        lax.fori_loop(0, NB, jbody, 0)
    pltpu.sync_copy(deg_t, out.at[pl.ds(wid * NKW * NP, NKW * NP)])


def _deg_call(dstT, wT5):
    f = pl.kernel(
        _deg_body,
        out_type=(jax.ShapeDtypeStruct((NT * NKW * NP,), jnp.float32),),
        mesh=_vsmesh(),
        compiler_params=pltpu.CompilerParams(needs_layout_passes=False),
        scratch_types=[
            pltpu.VMEM((NB, TB), jnp.int32),
            pltpu.VMEM((NB, TB), jnp.float32),
            pltpu.VMEM((NKW * NP,), jnp.float32),
        ],
    )
    return f(dstT, wT5)[0]


def _coef_body(srcT, dstT, wT5, dis5, out, src_v, dst_v, w_v, dis_v, co_v):
    c = lax.axis_index("c")
    s = lax.axis_index("s")
    wid = c * NS + s
    pltpu.sync_copy(srcT.at[pl.ds(wid * NB, NB)], src_v)
    pltpu.sync_copy(dstT.at[pl.ds(wid * NB, NB)], dst_v)
    for k in range(NKW):
        pltpu.sync_copy(dis5.at[pl.ds(k * NP, N)], dis_v.at[pl.ds(0, N)])
        pltpu.sync_copy(wT5.at[pl.ds((wid * NKW + k) * NB, NB)], w_v)

        def jbody(j, _):
            for g in range(8):
                sv = src_v[j, pl.ds(g * 16, 16)]
                dv = dst_v[j, pl.ds(g * 16, 16)]
                wv = w_v[j, pl.ds(g * 16, 16)]
                cs = plsc.load_gather(dis_v, [sv])
                cd = plsc.load_gather(dis_v, [dv])
                co_v[pl.ds(j * TB + g * 16, 16)] = cs * wv * cd
            return 0
        lax.fori_loop(0, NB, jbody, 0)
        pltpu.sync_copy(co_v,
                        out.at[pl.ds((wid * NKW + k) * NB * TB, NB * TB)])


def _coef_call(srcT, dstT, wT5, dis5):
    f = pl.kernel(
        _coef_body,
        out_type=(jax.ShapeDtypeStruct((NT * NKW * NB * TB,), jnp.float32),),
        mesh=_vsmesh(),
        compiler_params=pltpu.CompilerParams(needs_layout_passes=False),
        scratch_types=[
            pltpu.VMEM((NB, TB), jnp.int32),
            pltpu.VMEM((NB, TB), jnp.int32),
            pltpu.VMEM((NB, TB), jnp.float32),
            pltpu.VMEM((NP,), jnp.float32),
            pltpu.VMEM((NB * TB,), jnp.float32),
        ],
    )
    return f(srcT, dstT, wT5, dis5)[0]


NB_H = (48, 40)


def _agg_body(pairs, hs, srcT, dstT, coefT, out,
              src_v, dst_v, co_v, rows_v, yacc):
    c = lax.axis_index("c")
    s = lax.axis_index("s")
    wid = c * NS + s

    def zrow(i, _):
        for t in range(8):
            rows_v[i, pl.ds(t * 16, 16)] = jnp.zeros((16,), jnp.float32)
        return 0

    for pi, (hi, ki) in enumerate(pairs):
        lax.fori_loop(0, TB, zrow, 0)
        off = s * RPT
        for z0 in range(0, RPT, TB):
            pltpu.sync_copy(rows_v, yacc.at[pl.ds(off + z0, TB)])
        plsc.subcore_barrier()

        hoff = 0
        for nb2 in NB_H:
            pltpu.sync_copy(srcT.at[pl.ds(wid * NB + hoff, nb2)],
                            src_v.at[pl.ds(0, nb2)])
            pltpu.sync_copy(dstT.at[pl.ds(wid * NB + hoff, nb2)],
                            dst_v.at[pl.ds(0, nb2)])
            pltpu.sync_copy(
                coefT.at[pl.ds(((wid * NKW + ki) * NB + hoff) * TB,
                               nb2 * TB)],
                co_v.at[pl.ds(0, nb2 * TB)])
            hoff += nb2
            if hi:
                def obody(j, _):
                    def ogrp(g, _g):
                        src_v[j, pl.ds(g * 16, 16)] = (
                            src_v[j, pl.ds(g * 16, 16)] + hi * NP)
                        return 0
                    lax.fori_loop(0, 8, ogrp, 0)
                    return 0
                lax.fori_loop(0, nb2, obody, 0)

            def jbody(j, _):
                pltpu.sync_copy(hs.at[src_v.at[j]], rows_v)
                jj = _splat(j * TB)

                def gbody(g, _g):
                    base = g * 16
                    jb = jj + base
                    for r in range(16):
                        cr = plsc.load_gather(co_v, [jb + r])
                        for t in range(8):
                            rows_v[base + r, pl.ds(t * 16, 16)] = (
                                rows_v[base + r, pl.ds(t * 16, 16)] * cr)
                    return 0
                lax.fori_loop(0, 8, gbody, 0)
                pltpu.sync_copy(rows_v, yacc.at[dst_v.at[j]], add=True)
                return 0
            lax.fori_loop(0, nb2, jbody, 0)
        plsc.subcore_barrier()
        for z0 in range(0, RPT, TB):
            pltpu.sync_copy(yacc.at[pl.ds(s * RPT + z0, TB)], rows_v)
            pltpu.sync_copy(
                rows_v,
                out.at[pl.ds((pi * NC + c) * NP + s * RPT + z0, TB)])
        plsc.subcore_barrier()


def _agg_call(pairs, hs, srcT, dstT, coefT):
    npair = len(pairs)
    f = pl.kernel(
        functools.partial(_agg_body, pairs),
        out_type=(jax.ShapeDtypeStruct((npair * NC * NP, D), jnp.float32),),
        mesh=_vsmesh(),
        compiler_params=pltpu.CompilerParams(needs_layout_passes=False),
        scratch_types=[
            pltpu.VMEM((NB_H[0], TB), jnp.int32),
            pltpu.VMEM((NB_H[0], TB), jnp.int32),
            pltpu.VMEM((NB_H[0] * TB,), jnp.float32),
            pltpu.VMEM((TB, D), jnp.float32),
            pltpu.VMEM_SHARED((NP, D), jnp.float32),
        ],
    )
    return f(hs, srcT, dstT, coefT)[0]


def _prep_body(dp_ref, dis_ref):
    deg = jnp.sum(dp_ref[...], axis=0)
    dis_ref[...] = lax.rsqrt(deg)


def _prep_call(degp):
    return pl.pallas_call(
        _prep_body,
        out_shape=jax.ShapeDtypeStruct((NKW * NP,), jnp.float32),
    )(degp)


def _mm0_body(x_ref, w_ref, o_ref):
    o_ref[...] = jnp.dot(x_ref[...], w_ref[0],
                         preferred_element_type=jnp.float32)


def _mm0_call(xp, w1s):
    blk = 512
    return pl.pallas_call(
        _mm0_body,
        grid=(4, NP // blk),
        in_specs=[
            pl.BlockSpec((blk, D), lambda k, i: (i, 0)),
            pl.BlockSpec((1, D, D), lambda k, i: (k, 0, 0)),
        ],
        out_specs=pl.BlockSpec((blk, D), lambda k, i: (k * (NP // blk) + i, 0)),
        out_shape=jax.ShapeDtypeStruct((4 * NP, D), jnp.float32),
    )(xp, w1s)


def _mm2_body(p_ref, b_ref, w_ref, o_ref):
    p = p_ref[...]
    b = b_ref[...]
    zs = [jnp.maximum(p[2 * k] + p[2 * k + 1] + b[k:k + 1, :], 0.0)
          for k in range(4)]
    z = jnp.concatenate(zs, axis=1)
    h2 = jnp.dot(z, w_ref[...], preferred_element_type=jnp.float32)
    o_ref[0] = h2[:, :D]
    o_ref[1] = h2[:, D:]


def _mm2_call(part1, b1s, w2):
    blk = 512
    return pl.pallas_call(
        _mm2_body,
        grid=(NP // blk,),
        in_specs=[
            pl.BlockSpec((8, blk, D), lambda i: (0, i, 0)),
            pl.BlockSpec((8, D), lambda i: (0, 0)),
            pl.BlockSpec((4 * D, 2 * D), lambda i: (0, 0)),
        ],
        out_specs=pl.BlockSpec((2, blk, D), lambda i: (0, i, 0)),
        out_shape=jax.ShapeDtypeStruct((2, NP, D), jnp.float32),
    )(part1, b1s, w2)


def _mm3_body(p_ref, b_ref, w_ref, o_ref):
    p = p_ref[...]
    b = b_ref[...]
    x2a = jnp.maximum(p[0] + p[1] + b[0:1, :], 0.0)
    x2b = jnp.maximum(p[2] + p[3] + b[1:2, :], 0.0)
    o_ref[...] = (
        jnp.dot(x2a, w_ref[:D], preferred_element_type=jnp.float32)
        + jnp.dot(x2b, w_ref[D:], preferred_element_type=jnp.float32))


def _mm3_call(part2, b2r, w3):
    blk = 512
    return pl.pallas_call(
        _mm3_body,
        grid=(NP // blk,),
        in_specs=[
            pl.BlockSpec((4, blk, D), lambda i: (0, i, 0)),
            pl.BlockSpec((8, D), lambda i: (0, 0)),
            pl.BlockSpec((2 * D, D), lambda i: (0, 0)),
        ],
        out_specs=pl.BlockSpec((blk, D), lambda i: (i, 0)),
        out_shape=jax.ShapeDtypeStruct((NP, D), jnp.float32),
    )(part2, b2r, w3)


def _pool_body(p_ref, b_ref, bb_ref, sum_ref, max_ref, cnt_ref):
    i = pl.program_id(0)

    @pl.when(i == 0)
    def _():
        sum_ref[...] = jnp.zeros_like(sum_ref)
        cnt_ref[...] = jnp.zeros_like(cnt_ref)
        max_ref[...] = jnp.full_like(max_ref, -jnp.inf)

    x3 = p_ref[0] + p_ref[1] + b_ref[0:1, :]
    bb = bb_ref[...]
    gmin = jnp.min(bb)
    gmax = jnp.minimum(jnp.max(bb), G - 1)

    def gbody(g, _):
        m = bb == g
        xs = jnp.where(m, x3, 0.0)
        xm = jnp.where(m, x3, -jnp.inf)
        sum_ref[pl.ds(g, 1), :] += jnp.sum(xs, axis=0, keepdims=True)
        max_ref[pl.ds(g, 1), :] = jnp.maximum(
            max_ref[pl.ds(g, 1), :], jnp.max(xm, axis=0, keepdims=True))
        cnt_ref[pl.ds(g, 1), :] += jnp.sum(
            m.astype(jnp.float32), axis=0, keepdims=True)
        return 0
    lax.fori_loop(gmin, gmax + 1, gbody, 0)


def _pool_call(part3, b3r, batchb):
    blk = 512
    shp = jax.ShapeDtypeStruct((G, D), jnp.float32)
    return pl.pallas_call(
        _pool_body,
        grid=(NP // blk,),
        in_specs=[
            pl.BlockSpec((2, blk, D), lambda i: (0, i, 0)),
            pl.BlockSpec((8, D), lambda i: (0, 0)),
            pl.BlockSpec((blk, D), lambda i: (i, 0)),
        ],
        out_specs=(pl.BlockSpec((G, D), lambda i: (0, 0)),) * 3,
        out_shape=(shp, shp, shp),
    )(part3, b3r, batchb)


def _head_body(s0_ref, m0_ref, c0_ref, s1_ref, m1_ref, c1_ref,
               w1_ref, b1_ref, w2_ref, b2_ref, o_ref):
    mean0 = s0_ref[...] / jnp.maximum(c0_ref[...], 1.0)
    mean1 = s1_ref[...] / jnp.maximum(c1_ref[...], 1.0)
    p = jnp.concatenate([mean0, m0_ref[...], mean1, m1_ref[...]], axis=1)
    h = jnp.maximum(
        jnp.dot(p, w1_ref[...], preferred_element_type=jnp.float32)
        + b1_ref[0:1, :], 0.0)
    o_ref[...] = (jnp.dot(h, w2_ref[...], preferred_element_type=jnp.float32)
                  + b2_ref[0:1, :])


def _head_call(s0, m0, c0, s1, m1, c1, wm1p, bm1p, wm2p, bm2p):
    return pl.pallas_call(
        _head_body,
        out_shape=jax.ShapeDtypeStruct((G, D), jnp.float32),
    )(s0, m0, c0, s1, m1, c1, wm1p, bm1p, wm2p, bm2p)


def _edge_layout(ei, ea):
    loop = jnp.arange(N, dtype=jnp.int32)
    src = jnp.concatenate([ei[0], loop])
    dst = jnp.concatenate([ei[1], loop])
    w5 = jnp.concatenate([
        jnp.concatenate([ea, jnp.ones((E, 1), jnp.float32)], axis=1),
        jnp.ones((N, NKW), jnp.float32)], axis=0)
    pad = EPAD - EALL
    srcp = jnp.pad(src, (0, pad))
    dstp = jnp.pad(dst, (0, pad))
    w5p = jnp.pad(w5, ((0, pad), (0, 0)))

    def tileize(a):
        return (a.reshape(NKW, NT, NB, TB).transpose(1, 0, 2, 3)
                .reshape(NT * NKW * NB, TB))

    srcT = srcp.reshape(NT * NB, TB)
    dstT = dstp.reshape(NT * NB, TB)
    wT5 = tileize(w5p.T)
    return srcT, dstT, wT5


def _branch(x, ea, ei, batch, w1s, b1s, W2, b2r, W3, b3r):
    srcT, dstT, wT5 = _edge_layout(ei, ea)

    degp = _deg_call(dstT, wT5).reshape(NT, NKW * NP)
    dis5 = _prep_call(degp)
    coefT = _coef_call(srcT, dstT, wT5, dis5)

    xp = jnp.pad(x, ((0, NP - N), (0, 0)))
    h1s = _mm0_call(xp, w1s)
    part1 = _agg_call(((0, 0), (1, 1), (2, 2), (3, 3)),
                      h1s, srcT, dstT, coefT).reshape(8, NP, D)
    h2 = _mm2_call(part1, b1s, W2).reshape(2 * NP, D)
    part2 = _agg_call(((0, 4), (1, 4)),
                      h2, srcT, dstT, coefT).reshape(4, NP, D)
    h3 = _mm3_call(part2, b2r, W3)
    part3 = _agg_call(((0, 4),), h3, srcT, dstT, coefT).reshape(2, NP, D)

    batchp = jnp.pad(batch, (0, NP - N), constant_values=G)
    batchb = jnp.broadcast_to(batchp[:, None], (NP, D)).astype(jnp.int32)
    return _pool_call(part3, b3r, batchb)


def kernel(x0, edge_attr0, edge_index0, x1, edge_attr1, edge_index1,
           batch0, batch1,
           W1A, b1A, W1B, b1B, W1C, b1C, W1D, b1D, W2, b2, W3, b3,
           Wm1, bm1, Wm2, bm2):
    w1s = jnp.stack([W1A, W1B, W1C, W1D])
    b1s = jnp.pad(jnp.stack([b1A, b1B, b1C, b1D]), ((0, 4), (0, 0)))
    b2r = jnp.pad(b2.reshape(2, D), ((0, 6), (0, 0)))
    b3r = jnp.pad(b3.reshape(1, D), ((0, 7), (0, 0)))

    s0, m0, c0 = _branch(x0, edge_attr0, edge_index0, batch0,
                         w1s, b1s, W2, b2r, W3, b3r)
    s1, m1, c1 = _branch(x1, edge_attr1, edge_index1, batch1,
                         w1s, b1s, W2, b2r, W3, b3r)

    wm1p = jnp.pad(Wm1, ((0, 0), (0, 120)))
    bm1p = jnp.pad(bm1.reshape(1, 8), ((0, 7), (0, 120)))
    wm2p = jnp.pad(Wm2, ((0, 120), (0, 126)))
    bm2p = jnp.pad(bm2.reshape(1, 2), ((0, 7), (0, 126)))
    out = _head_call(s0, m0, c0, s1, m1, c1, wm1p, bm1p, wm2p, bm2p)
    return out[:, :2]
